# Initial kernel scaffold; baseline (speedup 1.0000x reference)
import jax
import jax.numpy as jnp
from jax.experimental import pallas as pl


def kernel(x, upsample_inds):
    raise NotImplementedError("write your pallas kernel here")


# speedup vs baseline: 1.0034x; 1.0034x over previous
import functools

import jax
import jax.numpy as jnp
from jax import lax
from jax.experimental import pallas as pl
from jax.experimental.pallas import tpu as pltpu
from jax.experimental.pallas import tpu_sc as plsc

_D = 128
_CHUNK = 128
_NW = 32


def _gather_body(nchunks, x_hbm, idx_hbm, out_hbm, idx_v, rows_v, sem):
  wid = lax.axis_index("s") * 2 + lax.axis_index("c")
  wbase = wid * (nchunks * _CHUNK)

  @pl.loop(0, nchunks)
  def _chunk(j):
    base = wbase + j * _CHUNK
    pltpu.sync_copy(idx_hbm.at[pl.ds(base, _CHUNK)], idx_v)
    pltpu.async_copy(x_hbm.at[idx_v], rows_v, sem).wait()
    pltpu.sync_copy(rows_v, out_hbm.at[pl.ds(base, _CHUNK)])


@functools.partial(jax.jit, static_argnums=(2,))
def _gather(x, idx_pad, nchunks):
  mesh = plsc.VectorSubcoreMesh(core_axis_name="c", subcore_axis_name="s")
  b_pad = idx_pad.shape[0]
  run = pl.kernel(
      functools.partial(_gather_body, nchunks),
      out_type=jax.ShapeDtypeStruct((b_pad, _D), jnp.float32),
      mesh=mesh,
      scratch_types=[
          pltpu.VMEM((_CHUNK,), jnp.int32),
          pltpu.VMEM((_CHUNK, _D), jnp.float32),
          pltpu.SemaphoreType.DMA,
      ],
  )
  return run(x, idx_pad)


def kernel(x, upsample_inds):
  n_out = upsample_inds.shape[0]
  idx = upsample_inds[:, 0].astype(jnp.int32)
  per_w = _NW * _CHUNK
  nchunks = (n_out + per_w - 1) // per_w
  b_pad = nchunks * per_w
  idx_pad = jnp.pad(idx, (0, b_pad - n_out))
  out = _gather(x, idx_pad, nchunks)
  return out[:n_out]


# speedup vs baseline: 1.5678x; 1.5624x over previous
import functools

import jax
import jax.numpy as jnp
from jax import lax
from jax.experimental import pallas as pl
from jax.experimental.pallas import tpu as pltpu
from jax.experimental.pallas import tpu_sc as plsc

_D = 128
_CHUNK = 128
_NW = 32


def _gather_body(nchunks, tail, x_hbm, idx_hbm, out_hbm, idx_v, rows_v, sem):
  wid = lax.axis_index("s") * 2 + lax.axis_index("c")
  nj = (nchunks - wid + _NW - 1) // _NW

  @pl.loop(0, nj)
  def _chunk(j):
    c = wid + j * _NW
    base = c * _CHUNK
    pltpu.sync_copy(idx_hbm.at[pl.ds(base, _CHUNK)], idx_v)
    pltpu.async_copy(x_hbm.at[idx_v], rows_v, sem).wait()
    if tail == _CHUNK:
      pltpu.sync_copy(rows_v, out_hbm.at[pl.ds(base, _CHUNK)])
    else:
      @pl.when(c != nchunks - 1)
      def _full():
        pltpu.sync_copy(rows_v, out_hbm.at[pl.ds(base, _CHUNK)])

      @pl.when(c == nchunks - 1)
      def _tail():
        pltpu.sync_copy(rows_v.at[pl.ds(0, tail)], out_hbm.at[pl.ds(base, tail)])


@functools.partial(jax.jit, static_argnums=(2, 3, 4))
def _gather(x, idx_pad, n_out, nchunks, tail):
  mesh = plsc.VectorSubcoreMesh(core_axis_name="c", subcore_axis_name="s")
  run = pl.kernel(
      functools.partial(_gather_body, nchunks, tail),
      out_type=jax.ShapeDtypeStruct((n_out, _D), jnp.float32),
      mesh=mesh,
      scratch_types=[
          pltpu.VMEM((_CHUNK,), jnp.int32),
          pltpu.VMEM((_CHUNK, _D), jnp.float32),
          pltpu.SemaphoreType.DMA,
      ],
  )
  return run(x, idx_pad)


def kernel(x, upsample_inds):
  n_out = upsample_inds.shape[0]
  idx = upsample_inds[:, 0].astype(jnp.int32)
  nchunks = (n_out + _CHUNK - 1) // _CHUNK
  tail = n_out - (nchunks - 1) * _CHUNK
  idx_pad = jnp.pad(idx, (0, nchunks * _CHUNK - n_out))
  return _gather(x, idx_pad, n_out, nchunks, tail)


# speedup vs baseline: 2.4813x; 1.5827x over previous
import functools

import jax
import jax.numpy as jnp
from jax import lax
from jax.experimental import pallas as pl
from jax.experimental.pallas import tpu as pltpu
from jax.experimental.pallas import tpu_sc as plsc

_D = 128
_CHUNK = 128
_NW = 32
_NB = 4


def _gather_body(nchunks, tail, x_hbm, idx_hbm, out_hbm, ibuf, r0, r1, r2, r3,
                 g0, g1, g2, g3, s0, s1, s2, s3):
  rows = [r0, r1, r2, r3]
  gsem = [g0, g1, g2, g3]
  ssem = [s0, s1, s2, s3]

  big = (nchunks + _NW - 1) // _NW
  cut = nchunks - (big - 1) * _NW

  w = lax.axis_index("s") * 2 + lax.axis_index("c")
  nj = jnp.where(w < cut, big, big - 1)
  base_chunk = jnp.where(w < cut, w * big, cut * big + (w - cut) * (big - 1))

  pltpu.sync_copy(idx_hbm.at[pl.ds(base_chunk * _CHUNK, big * _CHUNK)], ibuf)

  def start_gather(j, p):
    pltpu.async_copy(
        x_hbm.at[ibuf.at[pl.ds(j * _CHUNK, _CHUNK)]], rows[p], gsem[p])

  def wait_gather(p):
    pltpu.make_async_copy(
        x_hbm.at[ibuf.at[pl.ds(0, _CHUNK)]], rows[p], gsem[p]).wait()

  def start_store(j, p):
    gc = base_chunk + j
    if tail == _CHUNK:
      pltpu.async_copy(rows[p], out_hbm.at[pl.ds(gc * _CHUNK, _CHUNK)], ssem[p])
    else:
      @pl.when(gc == nchunks - 1)
      def _t():
        pltpu.async_copy(rows[p].at[pl.ds(0, tail)],
                         out_hbm.at[pl.ds(gc * _CHUNK, tail)], ssem[p])

      @pl.when(gc != nchunks - 1)
      def _f():
        pltpu.async_copy(rows[p], out_hbm.at[pl.ds(gc * _CHUNK, _CHUNK)],
                         ssem[p])

  def wait_store(p, is_tail):
    n = tail if is_tail else _CHUNK
    pltpu.make_async_copy(rows[p].at[pl.ds(0, n)],
                          out_hbm.at[pl.ds(0, n)], ssem[p]).wait()

  for p in range(_NB - 1):
    @pl.when(p < nj)
    def _prime(p=p):
      start_gather(p, p)

  nrounds = (nj + _NB - 1) // _NB

  @pl.loop(0, nrounds)
  def _round(r):
    for p in range(_NB):
      j = r * _NB + p

      @pl.when(j < nj)
      def _body(j=j, p=p):
        pm1 = (p - 1) % _NB

        @pl.when(j >= 1)
        def _drain_prev():
          wait_store(pm1, False)

        @pl.when(j + _NB - 1 < nj)
        def _prefetch():
          start_gather(j + _NB - 1, pm1)

        wait_gather(p)
        start_store(j, p)

  last_p = lax.rem(nj - 1, _NB)
  last_is_tail = (base_chunk + nj - 1) == (nchunks - 1)
  for p in range(_NB):
    @pl.when(last_p == p)
    def _drain_last(p=p):
      if tail == _CHUNK:
        wait_store(p, False)
      else:
        @pl.when(last_is_tail)
        def _t():
          wait_store(p, True)

        @pl.when(jnp.logical_not(last_is_tail))
        def _f():
          wait_store(p, False)


@functools.partial(jax.jit, static_argnums=(2, 3, 4))
def _gather(x, idx_pad, n_out, nchunks, tail):
  big = (nchunks + _NW - 1) // _NW
  mesh = plsc.VectorSubcoreMesh(core_axis_name="c", subcore_axis_name="s")
  run = pl.kernel(
      functools.partial(_gather_body, nchunks, tail),
      out_type=jax.ShapeDtypeStruct((n_out, _D), jnp.float32),
      mesh=mesh,
      scratch_types=[pltpu.VMEM((big * _CHUNK,), jnp.int32)]
      + [pltpu.VMEM((_CHUNK, _D), jnp.float32) for _ in range(_NB)]
      + [pltpu.SemaphoreType.DMA for _ in range(2 * _NB)],
  )
  return run(x, idx_pad)


def kernel(x, upsample_inds):
  n_out = upsample_inds.shape[0]
  idx = upsample_inds[:, 0].astype(jnp.int32)
  nchunks = (n_out + _CHUNK - 1) // _CHUNK
  tail = n_out - (nchunks - 1) * _CHUNK
  big = (nchunks + _NW - 1) // _NW
  cut = nchunks - (big - 1) * _NW
  last_base = cut * big + (_NW - 1 - cut) * (big - 1)
  pad_len = (last_base + big) * _CHUNK
  idx_pad = jnp.pad(idx, (0, pad_len - n_out))
  return _gather(x, idx_pad, n_out, nchunks, tail)
